# Initial kernel scaffold; baseline (speedup 1.0000x reference)
import jax
import jax.numpy as jnp
from jax.experimental import pallas as pl


def kernel(queries, keys, values, attn_mask, query_lengths, key_lengths, planes):
    raise NotImplementedError("write your pallas kernel here")


# speedup vs baseline: 13.7047x; 13.7047x over previous
import functools

import jax
import jax.numpy as jnp
import numpy as np
from jax.experimental import pallas as pl
from jax.experimental.pallas import tpu as pltpu

H, L, E = 12, 2048, 64
C = 100
CP = 128
BITS = 32
ITERS = 10
TOPK = 32
BQ = 256
NBQ = L // BQ
TEMP = 1.0 / np.sqrt(E).astype(np.float32)


def _cluster_kernel(q_ref, w_ref, b_ref, lab_ref, qg_ref):
    q = q_ref[0]
    h = jnp.dot(q.astype(jnp.bfloat16), w_ref[...].astype(jnp.bfloat16),
                preferred_element_type=jnp.float32)
    h = h + b_ref[0:1, :]
    bit = (h > 0).astype(jnp.float32)
    rs_bit = jnp.sum(bit, axis=1, keepdims=True)

    iota_c = jax.lax.broadcasted_iota(jnp.int32, (1, CP), 1)
    iota_l = jax.lax.broadcasted_iota(jnp.int32, (CP, L), 1)
    ci = jax.lax.broadcasted_iota(jnp.int32, (CP, 1), 0).astype(jnp.float32)
    init_idx = jnp.floor(ci * float(L) / float(C)).astype(jnp.int32)
    oh_init = (iota_l == init_idx).astype(jnp.float32)
    cbit0 = jnp.dot(oh_init, bit, preferred_element_type=jnp.float32,
                    precision=jax.lax.Precision.HIGHEST)
    invalid = (iota_c >= C).astype(jnp.float32) * 1e9

    def dist_labels(cbit):
        rs_c = jnp.sum(cbit, axis=1)[None, :]
        d = rs_bit + rs_c - 2.0 * jnp.dot(bit, cbit.T,
                                          preferred_element_type=jnp.float32,
                                          precision=jax.lax.Precision.HIGHEST)
        d = d + invalid
        m = jnp.min(d, axis=1, keepdims=True)
        cand = jnp.where(d == m, iota_c, CP)
        lab = jnp.min(cand, axis=1, keepdims=True)
        oh = (iota_c == lab).astype(jnp.float32)
        return lab, oh

    def body(_, cbit):
        _, oh = dist_labels(cbit)
        cnt = jnp.sum(oh, axis=0)[:, None]
        bcnt = jax.lax.dot_general(oh, bit, (((0,), (0,)), ((), ())),
                                   preferred_element_type=jnp.float32,
                                   precision=jax.lax.Precision.HIGHEST)
        newc = (2.0 * bcnt > cnt).astype(jnp.float32)
        return jnp.where(cnt > 0, newc, cbit)

    cbit = jax.lax.fori_loop(0, ITERS, body, cbit0)
    lab, oh = dist_labels(cbit)
    cnt = jnp.sum(oh, axis=0)[:, None]
    f = 1.0 / jnp.maximum(cnt, 1.0)
    qg = jax.lax.dot_general(oh, q, (((0,), (0,)), ((), ())),
                             preferred_element_type=jnp.float32,
                             precision=jax.lax.Precision.HIGHEST) * f
    lab_ref[0] = lab
    qg_ref[0] = qg


def _run_cluster(qt, w, b):
    return pl.pallas_call(
        _cluster_kernel,
        grid=(H,),
        in_specs=[
            pl.BlockSpec((1, L, E), lambda h: (h, 0, 0)),
            pl.BlockSpec((E, BITS), lambda h: (0, 0)),
            pl.BlockSpec((8, BITS), lambda h: (0, 0)),
        ],
        out_specs=[
            pl.BlockSpec((1, L, 1), lambda h: (h, 0, 0)),
            pl.BlockSpec((1, CP, E), lambda h: (h, 0, 0)),
        ],
        out_shape=[
            jax.ShapeDtypeStruct((H, L, 1), jnp.int32),
            jax.ShapeDtypeStruct((H, CP, E), jnp.float32),
        ],
    )(qt, w, b)


def _topk_kernel(qg_ref, k_ref, mask_ref, topi_ref):
    qg = qg_ref[0]
    k = k_ref[0]
    s = jax.lax.dot_general(qg.astype(jnp.bfloat16), k.astype(jnp.bfloat16),
                            (((1,), (1,)), ((), ())),
                            preferred_element_type=jnp.float32)
    s = s + mask_ref[0:1, :]
    iota_l = jax.lax.broadcasted_iota(jnp.int32, (CP, L), 1)
    for j in range(TOPK):
        m = jnp.max(s, axis=1, keepdims=True)
        cand = jnp.where(s == m, iota_l, L)
        idx = jnp.min(cand, axis=1, keepdims=True)
        topi_ref[0, :, j:j + 1] = idx
        s = jnp.where(iota_l == idx, -jnp.inf, s)


def _run_topk(qg, kt, maskvec):
    return pl.pallas_call(
        _topk_kernel,
        grid=(H,),
        in_specs=[
            pl.BlockSpec((1, CP, E), lambda h: (h, 0, 0)),
            pl.BlockSpec((1, L, E), lambda h: (h, 0, 0)),
            pl.BlockSpec((8, L), lambda h: (0, 0)),
        ],
        out_specs=pl.BlockSpec((1, CP, TOPK), lambda h: (h, 0, 0)),
        out_shape=jax.ShapeDtypeStruct((H, CP, TOPK), jnp.int32),
    )(qg, kt, maskvec)


def _attn_kernel(cb_ref, sq_ref, spos_ref, sc_ref, topi_ref, gk_ref, gv_ref,
                 out_ref):
    h = pl.program_id(0)
    b = pl.program_id(1)
    c_lo = cb_ref[h, b, 0]
    c_hi = cb_ref[h, b, 1]
    sq = sq_ref[0]
    sqb = sq.astype(jnp.bfloat16)
    spos = spos_ref[0]
    sc = sc_ref[0]

    def body(c, acc):
        kk = gk_ref[0, pl.ds(c * TOPK, TOPK), :]
        vv = gv_ref[0, pl.ds(c * TOPK, TOPK), :]
        ti = topi_ref[0, pl.ds(c, 1), :]
        s = jax.lax.dot_general(sqb, kk.astype(jnp.bfloat16),
                                (((1,), (1,)), ((), ())),
                                preferred_element_type=jnp.float32)
        future = ti > spos
        s = jnp.where(future, -1e7, s) * TEMP
        m = jnp.max(s, axis=1, keepdims=True)
        p = jnp.exp(s - m)
        a = p / jnp.sum(p, axis=1, keepdims=True)
        a = jnp.where(future, 0.0, a)
        a = a * (sc == c).astype(jnp.float32)
        return acc + jnp.dot(a.astype(jnp.bfloat16), vv.astype(jnp.bfloat16),
                             preferred_element_type=jnp.float32)

    acc = jax.lax.fori_loop(c_lo, c_hi + 1,body,
                            jnp.zeros((BQ, E), jnp.float32))
    out_ref[0] = acc


def _run_attn(cbounds, sq, spos3, sc3, topi, gk, gv):
    grid_spec = pltpu.PrefetchScalarGridSpec(
        num_scalar_prefetch=1,
        grid=(H, NBQ),
        in_specs=[
            pl.BlockSpec((1, BQ, E), lambda h, b, cb: (h, b, 0)),
            pl.BlockSpec((1, BQ, 1), lambda h, b, cb: (h * NBQ + b, 0, 0)),
            pl.BlockSpec((1, BQ, 1), lambda h, b, cb: (h * NBQ + b, 0, 0)),
            pl.BlockSpec((1, CP, TOPK), lambda h, b, cb: (h, 0, 0)),
            pl.BlockSpec((1, CP * TOPK, E), lambda h, b, cb: (h, 0, 0)),
            pl.BlockSpec((1, CP * TOPK, E), lambda h, b, cb: (h, 0, 0)),
        ],
        out_specs=pl.BlockSpec((1, BQ, E), lambda h, b, cb: (h, b, 0)),
    )
    return pl.pallas_call(
        _attn_kernel,
        grid_spec=grid_spec,
        out_shape=jax.ShapeDtypeStruct((H, L, E), jnp.float32),
    )(cbounds, sq, spos3, sc3, topi, gk, gv)


def kernel(queries, keys, values, attn_mask, query_lengths, key_lengths,
           planes):
    qt = jnp.transpose(queries, (0, 2, 1, 3)).reshape(H, L, E)
    kt = jnp.transpose(keys, (0, 2, 1, 3)).reshape(H, L, E)
    vt = jnp.transpose(values, (0, 2, 1, 3)).reshape(H, L, E)
    w = planes[:, :E].T
    b = jnp.broadcast_to(planes[:, E][None, :], (8, BITS)) + 0.0

    labels3, qg = _run_cluster(qt, w, b)
    labels = labels3[..., 0]

    maskvec = jnp.where(jnp.arange(L) < key_lengths, 0.0, -1e9)
    maskvec = jnp.broadcast_to(maskvec[None, :].astype(jnp.float32), (8, L)) + 0.0
    topi = _run_topk(qg, kt, maskvec)

    sorted_indx = jnp.argsort(labels, axis=-1).astype(jnp.int32)
    sorted_clusters = jnp.take_along_axis(labels, sorted_indx, axis=-1)
    sq = jnp.take_along_axis(qt, sorted_indx[:, :, None], axis=1)

    cb_lo = sorted_clusters[:, ::BQ]
    cb_hi = sorted_clusters[:, BQ - 1::BQ]
    cbounds = jnp.stack([cb_lo, cb_hi], axis=-1).astype(jnp.int32)

    ti_flat = topi.reshape(H, CP * TOPK)
    gk = jnp.take_along_axis(kt, ti_flat[:, :, None], axis=1)
    gv = jnp.take_along_axis(vt, ti_flat[:, :, None], axis=1)

    spos3 = sorted_indx.reshape(H * NBQ, BQ, 1)
    sc3 = sorted_clusters.reshape(H * NBQ, BQ, 1).astype(jnp.int32)

    out_s = _run_attn(cbounds, sq, spos3, sc3, topi, gk, gv)

    rev = jnp.argsort(sorted_indx, axis=-1)
    out = jnp.take_along_axis(out_s, rev[:, :, None], axis=1)
    out = jnp.transpose(out.reshape(1, H, L, E), (0, 2, 1, 3))
    causal_ok = attn_mask != 0
    return jnp.where(causal_ok, out, jnp.full_like(out, jnp.nan))
